# baseline (speedup 1.0000x reference)
import functools
import math

import jax
import jax.numpy as jnp
from jax import lax
from jax.experimental import pallas as pl
from jax.experimental.pallas import tpu as pltpu
from jax.experimental.pallas import tpu_sc as plsc

N = 10000
E = 160000
H = 128
H3 = 3 * H

NC = 2
NS = 16
LN = 16

NPASS = 10
RANGE = 512
LAST = N - (NC * NPASS - 1) * RANGE
PAD = 16
EPT = E // NS
CHUNK = 32

INV3 = 1.0 / math.sqrt(3.0)
INVH = 1.0 / math.sqrt(float(H))


def _xh_body(x_ref, lnw_ref, lnb_ref, w1_ref, b1_ref, w2_ref, b2_ref, o_ref):
    x = x_ref[...]
    mu = jnp.mean(x, axis=-1, keepdims=True)
    xc = x - mu
    var = jnp.mean(xc * xc, axis=-1, keepdims=True)
    xn = xc * lax.rsqrt(var + 1e-5) * lnw_ref[...] + lnb_ref[...]
    h = jnp.dot(xn, w1_ref[...], preferred_element_type=jnp.float32) + b1_ref[...]
    h = (jax.nn.sigmoid(h) * h) * (1.0 / 0.6)
    o_ref[...] = jnp.dot(h, w2_ref[...], preferred_element_type=jnp.float32) + b2_ref[...]


_XB = 2000

_xh_call = pl.pallas_call(
    _xh_body,
    grid=(N // _XB,),
    in_specs=[
        pl.BlockSpec((_XB, H), lambda i: (i, 0)),
        pl.BlockSpec((1, H), lambda i: (0, 0)),
        pl.BlockSpec((1, H), lambda i: (0, 0)),
        pl.BlockSpec((H, H), lambda i: (0, 0)),
        pl.BlockSpec((1, H), lambda i: (0, 0)),
        pl.BlockSpec((H, H3), lambda i: (0, 0)),
        pl.BlockSpec((1, H3), lambda i: (0, 0)),
    ],
    out_specs=pl.BlockSpec((_XB, H3), lambda i: (i, 0)),
    out_shape=jax.ShapeDtypeStruct((N, H3), jnp.float32),
)


def _rbf_body(r_ref, wr_ref, br_ref, o_ref):
    o_ref[...] = (
        jnp.dot(r_ref[...], wr_ref[...], preferred_element_type=jnp.float32)
        + br_ref[...]
    )


_RB = 2000

_rbf_call = pl.pallas_call(
    _rbf_body,
    grid=(E // _RB,),
    in_specs=[
        pl.BlockSpec((_RB, H), lambda i: (i, 0)),
        pl.BlockSpec((H, H3), lambda i: (0, 0)),
        pl.BlockSpec((1, H3), lambda i: (0, 0)),
    ],
    out_specs=pl.BlockSpec((_RB, H3), lambda i: (i, 0)),
    out_shape=jax.ShapeDtypeStruct((E, H3), jnp.float32),
)


def _sc_body(src_hbm, dst_hbm, xh_hbm, vec_hbm, rbfh_hbm, ev_hbm,
             dx_hbm, dv_hbm,
             ssrc, sdst, eids, srcs, ldst,
             rbfb0, rbfb1, xhb0, xhb1, vecb0, vecb1, evb0, evb1,
             odx0, odx1, odv0, odv1,
             zdx, zdv, acc_dx, acc_dv,
             gsem0, gsem1, ssem0, ssem1):
    c = lax.axis_index("c")
    s = lax.axis_index("s")
    e0 = s * EPT

    rbfb = (rbfb0, rbfb1)
    xhb = (xhb0, xhb1)
    vecb = (vecb0, vecb1)
    evb = (evb0, evb1)
    odx = (odx0, odx1)
    odv = (odv0, odv1)
    gsem = (gsem0, gsem1)
    ssem = (ssem0, ssem1)

    pltpu.sync_copy(src_hbm.at[pl.ds(e0, EPT)], ssrc)
    pltpu.sync_copy(dst_hbm.at[pl.ds(e0, EPT)], sdst)

    zero = jnp.zeros((LN,), jnp.float32)

    def _zrow(r, _):
        for j in range(H // LN):
            zdx[r, pl.ds(LN * j, LN)] = zero
        for j in range(H3 // LN):
            zdv[r, pl.ds(LN * j, LN)] = zero
        return 0

    lax.fori_loop(0, LN, _zrow, 0)

    def _issue(k, b):
        kk = k * LN
        ei = eids[pl.ds(kk, LN)]
        si = srcs[pl.ds(kk, LN)]
        pltpu.async_copy(rbfh_hbm.at[ei], rbfb[b], gsem[b])
        pltpu.async_copy(xh_hbm.at[si], xhb[b], gsem[b])
        pltpu.async_copy(vec_hbm.at[si], vecb[b], gsem[b])
        pltpu.async_copy(ev_hbm.at[ei], evb[b], gsem[b])

    def _wait_gathers(b):
        pltpu.make_async_copy(rbfh_hbm.at[pl.ds(0, LN)], rbfb[b], gsem[b]).wait()
        pltpu.make_async_copy(xh_hbm.at[pl.ds(0, LN)], xhb[b], gsem[b]).wait()
        pltpu.make_async_copy(vec_hbm.at[pl.ds(0, LN)], vecb[b], gsem[b]).wait()
        pltpu.make_async_copy(ev_hbm.at[pl.ds(0, LN)], evb[b], gsem[b]).wait()

    def _wait_scatters(b):
        pltpu.make_async_copy(odx[b], acc_dx.at[pl.ds(0, LN)], ssem[b]).wait()
        pltpu.make_async_copy(odv[b], acc_dv.at[pl.ds(0, LN)], ssem[b]).wait()

    def _compute(b):
        def _edge(e, _):
            for j in range(H // LN):
                odx[b][e, pl.ds(LN * j, LN)] = (
                    rbfb[b][e, pl.ds(LN * j, LN)] * xhb[b][e, pl.ds(LN * j, LN)]
                )
            ve = evb[b][e, pl.ds(0, LN)]
            ev0 = ve[0]
            ev1 = ve[1]
            ev2 = ve[2]
            for j in range(H // LN):
                x2 = (
                    rbfb[b][e, pl.ds(H + LN * j, LN)]
                    * xhb[b][e, pl.ds(H + LN * j, LN)]
                    * INV3
                )
                x3 = (
                    rbfb[b][e, pl.ds(2 * H + LN * j, LN)]
                    * xhb[b][e, pl.ds(2 * H + LN * j, LN)]
                )
                for d, evd in enumerate((ev0, ev1, ev2)):
                    odv[b][e, pl.ds(d * H + LN * j, LN)] = (
                        vecb[b][e, pl.ds(d * H + LN * j, LN)] * x2 + evd * x3
                    ) * INVH
            return 0

        lax.fori_loop(0, LN, _edge, 0)

    def _pass(rpass, _):
        base = (c * NPASS + rpass) * RANGE

        def _zacc(i, _):
            row = jnp.minimum(s * CHUNK + i * LN, RANGE - LN)
            pltpu.sync_copy(zdx, acc_dx.at[pl.ds(row, LN)])
            pltpu.sync_copy(zdv, acc_dv.at[pl.ds(row, LN)])
            return 0

        lax.fori_loop(0, CHUNK // LN, _zacc, 0)
        plsc.subcore_barrier()

        def _scan(j, cnt):
            dv = sdst[pl.ds(j * LN, LN)]
            sv = ssrc[pl.ds(j * LN, LN)]
            m = (dv >= base) & (dv < base + RANGE)
            eid = (e0 + j * LN) + lax.iota(jnp.int32, 16)
            pos = cnt + plsc.cumsum(m.astype(jnp.int32)) - 1
            plsc.store_scatter(eids, [pos], eid, mask=m)
            plsc.store_scatter(srcs, [pos], sv, mask=m)
            plsc.store_scatter(ldst, [pos], dv - base, mask=m)
            return pos[LN - 1] + 1

        cnt = lax.fori_loop(0, EPT // LN, _scan, jnp.int32(0))

        for p in range(4):
            eids[pl.ds(cnt + p * LN, LN)] = jnp.zeros((LN,), jnp.int32)
            srcs[pl.ds(cnt + p * LN, LN)] = jnp.zeros((LN,), jnp.int32)
            ldst[pl.ds(cnt + p * LN, LN)] = jnp.full((LN,), RANGE, jnp.int32)

        nb = lax.div(cnt + (LN - 1), LN)
        pad_i = jnp.full((LN,), RANGE, jnp.int32)

        _issue(0, 0)
        _issue(1, 1)
        for b in range(2):
            pltpu.async_copy(odx[b], acc_dx.at[pad_i], ssem[b], add=True)
            pltpu.async_copy(odv[b], acc_dv.at[pad_i], ssem[b], add=True)

        @pl.loop(0, lax.div(nb + 1, jnp.int32(2)))
        def _outer(i):
            for b in range(2):
                k = i * 2 + b
                _wait_scatters(b)
                _wait_gathers(b)
                _compute(b)
                dv_i = ldst[pl.ds(k * LN, LN)]
                pltpu.async_copy(odx[b], acc_dx.at[dv_i], ssem[b], add=True)
                pltpu.async_copy(odv[b], acc_dv.at[dv_i], ssem[b], add=True)
                _issue(k + 2, b)
        for b in range(2):
            _wait_scatters(b)
            _wait_gathers(b)
        plsc.subcore_barrier()

        def _copy_out(size):
            row0 = jnp.minimum(s * CHUNK, size - CHUNK)
            pltpu.sync_copy(
                acc_dx.at[pl.ds(row0, CHUNK)], dx_hbm.at[pl.ds(base + row0, CHUNK)]
            )
            pltpu.sync_copy(
                acc_dv.at[pl.ds(row0, CHUNK)], dv_hbm.at[pl.ds(base + row0, CHUNK)]
            )

        is_last = (rpass == NPASS - 1) & (c == 1)
        pl.when(jnp.logical_not(is_last))(lambda: _copy_out(RANGE))
        pl.when(is_last)(lambda: _copy_out(LAST))
        plsc.subcore_barrier()
        return 0

    lax.fori_loop(0, NPASS, _pass, 0)


_sc_call = functools.partial(
    pl.kernel,
    out_type=(
        jax.ShapeDtypeStruct((N, H), jnp.float32),
        jax.ShapeDtypeStruct((N, H3), jnp.float32),
    ),
    mesh=plsc.VectorSubcoreMesh(core_axis_name="c", subcore_axis_name="s"),
    compiler_params=pltpu.CompilerParams(
        use_tc_tiling_on_sc=False, needs_layout_passes=False
    ),
    scratch_types=(
        pltpu.VMEM((EPT,), jnp.int32),
        pltpu.VMEM((EPT,), jnp.int32),
        pltpu.VMEM((EPT + 4 * LN,), jnp.int32),
        pltpu.VMEM((EPT + 4 * LN,), jnp.int32),
        pltpu.VMEM((EPT + 4 * LN,), jnp.int32),
        pltpu.VMEM((LN, H3), jnp.float32),
        pltpu.VMEM((LN, H3), jnp.float32),
        pltpu.VMEM((LN, H3), jnp.float32),
        pltpu.VMEM((LN, H3), jnp.float32),
        pltpu.VMEM((LN, H3), jnp.float32),
        pltpu.VMEM((LN, H3), jnp.float32),
        pltpu.VMEM((LN, LN), jnp.float32),
        pltpu.VMEM((LN, LN), jnp.float32),
        pltpu.VMEM((LN, H), jnp.float32),
        pltpu.VMEM((LN, H), jnp.float32),
        pltpu.VMEM((LN, H3), jnp.float32),
        pltpu.VMEM((LN, H3), jnp.float32),
        pltpu.VMEM((LN, H), jnp.float32),
        pltpu.VMEM((LN, H3), jnp.float32),
        pltpu.VMEM_SHARED((RANGE + PAD, H), jnp.float32),
        pltpu.VMEM_SHARED((RANGE + PAD, H3), jnp.float32),
        pltpu.SemaphoreType.DMA,
        pltpu.SemaphoreType.DMA,
        pltpu.SemaphoreType.DMA,
        pltpu.SemaphoreType.DMA,
    ),
)(_sc_body)


def kernel(x, vec, edge_index, edge_rbf, edge_vector, ln_w, ln_b, W1, b1, W2, b2, Wr, br):
    xh = _xh_call(
        x,
        ln_w.reshape(1, H),
        ln_b.reshape(1, H),
        W1,
        b1.reshape(1, H),
        W2,
        b2.reshape(1, H3),
    )
    rbfh = _rbf_call(edge_rbf, Wr, br.reshape(1, H3))
    src = edge_index[0]
    dst = edge_index[1]
    vecf = vec.reshape(N, H3)
    ev16 = jnp.pad(edge_vector, ((0, 0), (0, LN - 3)))
    dx, dvf = _sc_call(src, dst, xh, vecf, rbfh, ev16)
    return dx, dvf.reshape(N, 3, H)


# speedup vs baseline: 1.0887x; 1.0887x over previous
import functools
import math

import jax
import jax.numpy as jnp
from jax import lax
from jax.experimental import pallas as pl
from jax.experimental.pallas import tpu as pltpu
from jax.experimental.pallas import tpu_sc as plsc

N = 10000
E = 160000
H = 128
H3 = 3 * H

NC = 2
NS = 16
LN = 16

NPASS = 4
RANGE = 1264
LAST = N - (NC * NPASS - 1) * RANGE
PAD = 16
EPT = E // NS
CHUNK = 80

INV3 = 1.0 / math.sqrt(3.0)
INVH = 1.0 / math.sqrt(float(H))


def _xh_body(x_ref, lnw_ref, lnb_ref, w1_ref, b1_ref, w2_ref, b2_ref, o_ref):
    x = x_ref[...]
    mu = jnp.mean(x, axis=-1, keepdims=True)
    xc = x - mu
    var = jnp.mean(xc * xc, axis=-1, keepdims=True)
    xn = xc * lax.rsqrt(var + 1e-5) * lnw_ref[...] + lnb_ref[...]
    h = jnp.dot(xn, w1_ref[...], preferred_element_type=jnp.float32) + b1_ref[...]
    h = (jax.nn.sigmoid(h) * h) * (1.0 / 0.6)
    o_ref[...] = jnp.dot(h, w2_ref[...], preferred_element_type=jnp.float32) + b2_ref[...]


_XB = 2000

_xh_call = pl.pallas_call(
    _xh_body,
    grid=(N // _XB,),
    in_specs=[
        pl.BlockSpec((_XB, H), lambda i: (i, 0)),
        pl.BlockSpec((1, H), lambda i: (0, 0)),
        pl.BlockSpec((1, H), lambda i: (0, 0)),
        pl.BlockSpec((H, H), lambda i: (0, 0)),
        pl.BlockSpec((1, H), lambda i: (0, 0)),
        pl.BlockSpec((H, H3), lambda i: (0, 0)),
        pl.BlockSpec((1, H3), lambda i: (0, 0)),
    ],
    out_specs=pl.BlockSpec((_XB, H3), lambda i: (i, 0)),
    out_shape=jax.ShapeDtypeStruct((N, H3), jnp.float32),
)


def _rbf_body(r_ref, wr_ref, br_ref, o_ref):
    o_ref[...] = (
        jnp.dot(r_ref[...], wr_ref[...], preferred_element_type=jnp.float32)
        + br_ref[...]
    )


_RB = 2000

_rbf_call = pl.pallas_call(
    _rbf_body,
    grid=(E // _RB,),
    in_specs=[
        pl.BlockSpec((_RB, H), lambda i: (i, 0)),
        pl.BlockSpec((H, H3), lambda i: (0, 0)),
        pl.BlockSpec((1, H3), lambda i: (0, 0)),
    ],
    out_specs=pl.BlockSpec((_RB, H3), lambda i: (i, 0)),
    out_shape=jax.ShapeDtypeStruct((E, H3), jnp.float32),
)


def _sc_body(src_hbm, dst_hbm, xh_hbm, vec_hbm, rbfh_hbm, ev_hbm,
             dx_hbm, dv_hbm,
             ssrc, sdst, eids, srcs, ldst,
             rbfb, xhb, vecb, evb, odx, odv,
             zdx, zdv, acc_dx, acc_dv, sem):
    c = lax.axis_index("c")
    s = lax.axis_index("s")
    e0 = s * EPT

    pltpu.sync_copy(src_hbm.at[pl.ds(e0, EPT)], ssrc)
    pltpu.sync_copy(dst_hbm.at[pl.ds(e0, EPT)], sdst)

    zero = jnp.zeros((LN,), jnp.float32)

    def _zrow(r, _):
        for j in range(H // LN):
            zdx[r, pl.ds(LN * j, LN)] = zero
        for j in range(H3 // LN):
            zdv[r, pl.ds(LN * j, LN)] = zero
        return 0

    lax.fori_loop(0, LN, _zrow, 0)

    for rpass in range(NPASS):
        base = (c * NPASS + rpass) * RANGE

        def _zacc(i, _):
            row = jnp.minimum(s * CHUNK + i * LN, RANGE - LN)
            pltpu.sync_copy(zdx, acc_dx.at[pl.ds(row, LN)])
            pltpu.sync_copy(zdv, acc_dv.at[pl.ds(row, LN)])
            return 0

        lax.fori_loop(0, CHUNK // LN, _zacc, 0)
        plsc.subcore_barrier()

        def _scan(j, cnt):
            dv = sdst[pl.ds(j * LN, LN)]
            sv = ssrc[pl.ds(j * LN, LN)]
            m = (dv >= base) & (dv < base + RANGE)
            eid = (e0 + j * LN) + lax.iota(jnp.int32, 16)
            pos = cnt + plsc.cumsum(m.astype(jnp.int32)) - 1
            plsc.store_scatter(eids, [pos], eid, mask=m)
            plsc.store_scatter(srcs, [pos], sv, mask=m)
            plsc.store_scatter(ldst, [pos], dv - base, mask=m)
            return pos[LN - 1] + 1

        cnt = lax.fori_loop(0, EPT // LN, _scan, jnp.int32(0))

        eids[pl.ds(cnt, LN)] = jnp.zeros((LN,), jnp.int32)
        srcs[pl.ds(cnt, LN)] = jnp.zeros((LN,), jnp.int32)
        ldst[pl.ds(cnt, LN)] = jnp.full((LN,), RANGE, jnp.int32)

        def _batch(k, _):
            kk = k * LN
            ev_i = eids[pl.ds(kk, LN)]
            sv_i = srcs[pl.ds(kk, LN)]
            dv_i = ldst[pl.ds(kk, LN)]
            c1 = pltpu.async_copy(rbfh_hbm.at[ev_i], rbfb, sem)
            c2 = pltpu.async_copy(xh_hbm.at[sv_i], xhb, sem)
            c3 = pltpu.async_copy(vec_hbm.at[sv_i], vecb, sem)
            c4 = pltpu.async_copy(ev_hbm.at[ev_i], evb, sem)
            c1.wait()
            c2.wait()
            c3.wait()
            c4.wait()

            def _edge(e, _):
                for j in range(H // LN):
                    odx[e, pl.ds(LN * j, LN)] = (
                        rbfb[e, pl.ds(LN * j, LN)] * xhb[e, pl.ds(LN * j, LN)]
                    )
                ve = evb[e, pl.ds(0, LN)]
                ev0 = ve[0]
                ev1 = ve[1]
                ev2 = ve[2]
                for j in range(H // LN):
                    x2 = (
                        rbfb[e, pl.ds(H + LN * j, LN)]
                        * xhb[e, pl.ds(H + LN * j, LN)]
                    )
                    x3 = (
                        rbfb[e, pl.ds(2 * H + LN * j, LN)]
                        * xhb[e, pl.ds(2 * H + LN * j, LN)]
                    )
                    for d, evd in enumerate((ev0, ev1, ev2)):
                        odv[e, pl.ds(d * H + LN * j, LN)] = (
                            vecb[e, pl.ds(d * H + LN * j, LN)] * x2 + evd * x3
                        )
                return 0

            lax.fori_loop(0, LN, _edge, 0)
            pltpu.sync_copy(odx, acc_dx.at[dv_i], add=True)
            pltpu.sync_copy(odv, acc_dv.at[dv_i], add=True)
            return 0

        nb = lax.div(cnt + (LN - 1), LN)
        lax.fori_loop(0, nb, _batch, 0)
        plsc.subcore_barrier()

        def _copy_out(size):
            row0 = jnp.minimum(s * CHUNK, size - CHUNK)
            pltpu.sync_copy(
                acc_dx.at[pl.ds(row0, CHUNK)], dx_hbm.at[pl.ds(base + row0, CHUNK)]
            )
            pltpu.sync_copy(
                acc_dv.at[pl.ds(row0, CHUNK)], dv_hbm.at[pl.ds(base + row0, CHUNK)]
            )

        if rpass == NPASS - 1:
            pl.when(c == 0)(lambda: _copy_out(RANGE))
            pl.when(c == 1)(lambda: _copy_out(LAST))
        else:
            _copy_out(RANGE)
        plsc.subcore_barrier()


_sc_call = functools.partial(
    pl.kernel,
    out_type=(
        jax.ShapeDtypeStruct((N, H), jnp.float32),
        jax.ShapeDtypeStruct((N, H3), jnp.float32),
    ),
    mesh=plsc.VectorSubcoreMesh(core_axis_name="c", subcore_axis_name="s"),
    compiler_params=pltpu.CompilerParams(
        use_tc_tiling_on_sc=False, needs_layout_passes=False
    ),
    scratch_types=(
        pltpu.VMEM((EPT,), jnp.int32),
        pltpu.VMEM((EPT,), jnp.int32),
        pltpu.VMEM((EPT + LN,), jnp.int32),
        pltpu.VMEM((EPT + LN,), jnp.int32),
        pltpu.VMEM((EPT + LN,), jnp.int32),
        pltpu.VMEM((LN, H3), jnp.float32),
        pltpu.VMEM((LN, H3), jnp.float32),
        pltpu.VMEM((LN, H3), jnp.float32),
        pltpu.VMEM((LN, LN), jnp.float32),
        pltpu.VMEM((LN, H), jnp.float32),
        pltpu.VMEM((LN, H3), jnp.float32),
        pltpu.VMEM((LN, H), jnp.float32),
        pltpu.VMEM((LN, H3), jnp.float32),
        pltpu.VMEM_SHARED((RANGE + PAD, H), jnp.float32),
        pltpu.VMEM_SHARED((RANGE + PAD, H3), jnp.float32),
        pltpu.SemaphoreType.DMA,
    ),
)(_sc_body)


def kernel(x, vec, edge_index, edge_rbf, edge_vector, ln_w, ln_b, W1, b1, W2, b2, Wr, br):
    xh = _xh_call(
        x,
        ln_w.reshape(1, H),
        ln_b.reshape(1, H),
        W1,
        b1.reshape(1, H),
        W2,
        b2.reshape(1, H3),
    )
    scale = jnp.concatenate(
        (
            jnp.ones((H,), jnp.float32),
            jnp.full((H,), INV3 * INVH, jnp.float32),
            jnp.full((H,), INVH, jnp.float32),
        )
    )
    rbfh = _rbf_call(edge_rbf, Wr * scale, (br * scale).reshape(1, H3))
    src = edge_index[0]
    dst = edge_index[1]
    vecf = vec.reshape(N, H3)
    ev16 = jnp.pad(edge_vector, ((0, 0), (0, LN - 3)))
    dx, dvf = _sc_call(src, dst, xh, vecf, rbfh, ev16)
    return dx, dvf.reshape(N, 3, H)
